# baseline (speedup 1.0000x reference)
import functools

import jax
import jax.numpy as jnp
from jax import lax
from jax.experimental import pallas as pl
from jax.experimental.pallas import tpu as pltpu
from jax.experimental.pallas import tpu_sc as plsc

L = 16
NC = 2
NS = 16
NW = NC * NS
CHUNK = 64
NSLOT = 6
KUNROLL = 8


def _sc_fm(T, n_intrs, vec_size, n_segs):
    tpw = T // NW
    nchunk = tpw // CHUNK
    nblk = CHUNK // L

    mesh = plsc.VectorSubcoreMesh(
        core_axis_name="c", subcore_axis_name="s",
        num_cores=NC, num_subcores=NS)
    f32, i32 = jnp.float32, jnp.int32

    scratch = (
        [pltpu.VMEM((CHUNK,), i32) for _ in range(2 * NSLOT)]
        + [pltpu.VMEM((CHUNK, vec_size), f32) for _ in range(2 * NSLOT)]
        + [
            pltpu.VMEM((tpw,), i32),
            pltpu.VMEM((tpw,), f32),
            pltpu.VMEM((tpw,), i32),
            pltpu.VMEM((n_intrs,), f32),
            pltpu.VMEM((L,), f32),
            pltpu.VMEM((tpw,), f32),
            pltpu.VMEM((n_segs * L,), f32),
            pltpu.VMEM((L,), f32),
            pltpu.VMEM((NS, L), f32),
            pltpu.VMEM((L,), f32),
            pltpu.VMEM_SHARED((NS, L), f32),
        ]
        + [pltpu.SemaphoreType.DMA for _ in range(NSLOT + 1)]
    )

    @functools.partial(
        pl.kernel,
        out_type=jax.ShapeDtypeStruct((NC, n_segs), f32),
        mesh=mesh,
        compiler_params=pltpu.CompilerParams(needs_layout_passes=False),
        scratch_types=scratch,
    )
    def body(fa_h, fb_h, iv_h, dv_h, sg_h, vecs_h, wv_h, b_h, out_h, *scr):
        ias = scr[0:NSLOT]
        ibs = scr[NSLOT:2 * NSLOT]
        bufs_a = scr[2 * NSLOT:3 * NSLOT]
        bufs_b = scr[3 * NSLOT:4 * NSLOT]
        (iv_v, dv_v, sg_v, wv_v, b_v, d_v, accm_v, res_v, merge_v,
         res2_v, shared) = scr[4 * NSLOT:4 * NSLOT + 11]
        sems = scr[4 * NSLOT + 11:4 * NSLOT + 11 + NSLOT]
        sem_st = scr[4 * NSLOT + 11 + NSLOT]

        cid = lax.axis_index("c")
        sid = lax.axis_index("s")
        wid = cid * NS + sid
        base = wid * tpw
        iota = lax.iota(i32, L)

        def issue(c):
            slot = c % NSLOT
            off = base + c * CHUNK
            pltpu.sync_copy(fa_h.at[pl.ds(off, CHUNK)], ias[slot])
            pltpu.sync_copy(fb_h.at[pl.ds(off, CHUNK)], ibs[slot])
            da = pltpu.async_copy(vecs_h.at[ias[slot]], bufs_a[slot], sems[slot])
            db = pltpu.async_copy(vecs_h.at[ibs[slot]], bufs_b[slot], sems[slot])
            return da, db

        def compute(c, bufa, bufb):
            def blk_body(b, _):
                rows = b * L + iota

                def k_body(i, accs):
                    a0, a1 = accs
                    for u in range(KUNROLL):
                        kk = i * KUNROLL + u
                        col = (iota + kk) & (vec_size - 1)
                        va = plsc.load_gather(bufa, [rows, col])
                        vb = plsc.load_gather(bufb, [rows, col])
                        p = va * vb
                        if u % 2 == 0:
                            a0 = a0 + p
                        else:
                            a1 = a1 + p
                    return a0, a1

                z = jnp.zeros((L,), f32)
                a0, a1 = lax.fori_loop(0, vec_size // KUNROLL, k_body, (z, z))
                d_v[pl.ds(c * CHUNK + b * L, L)] = a0 + a1
                return 0

            lax.fori_loop(0, nblk, blk_body, 0)

        descs = {}
        for c in range(NSLOT - 1):
            descs[c] = issue(c)
        st = [
            pltpu.async_copy(iv_h.at[pl.ds(base, tpw)], iv_v, sem_st),
            pltpu.async_copy(dv_h.at[pl.ds(base, tpw)], dv_v, sem_st),
            pltpu.async_copy(sg_h.at[pl.ds(base, tpw)], sg_v, sem_st),
            pltpu.async_copy(wv_h, wv_v, sem_st),
            pltpu.async_copy(b_h, b_v, sem_st),
        ]
        for c in range(nchunk):
            if c + NSLOT - 1 < nchunk:
                descs[c + NSLOT - 1] = issue(c + NSLOT - 1)
            da, db = descs.pop(c)
            da.wait()
            db.wait()
            compute(c, bufs_a[c % NSLOT], bufs_b[c % NSLOT])

        for d_ in st:
            d_.wait()
        plsc.subcore_barrier()
        bvec = b_v[...]

        def seg_body(i, accs):
            t0 = i * L
            ivec = iv_v[pl.ds(t0, L)]
            w = plsc.load_gather(wv_v, [ivec])
            dv = dv_v[pl.ds(t0, L)]
            d = d_v[pl.ds(t0, L)]
            sg = sg_v[pl.ds(t0, L)]
            wdb = w / dv * d + bvec
            return tuple(
                accs[s] + jnp.where(sg == s, wdb, 0.0) for s in range(n_segs))

        zeros = tuple(jnp.zeros((L,), f32) for _ in range(n_segs))
        accs = lax.fori_loop(0, tpw // L, seg_body, zeros)
        for s in range(n_segs):
            accm_v[pl.ds(s * L, L)] = accs[s]

        plsc.subcore_barrier()
        res = jnp.zeros((L,), f32)
        for l in range(L):
            res = res + plsc.load_gather(accm_v, [iota * L + l])
        res_v[...] = res
        plsc.subcore_barrier()

        pltpu.sync_copy(res_v, shared.at[sid])
        plsc.subcore_barrier()

        @pl.when(sid == 0)
        def _():
            pltpu.sync_copy(shared, merge_v)
            m = merge_v[0, :]
            for s in range(1, NS):
                m = m + merge_v[s, :]
            res2_v[...] = m

        plsc.subcore_barrier()

        @pl.when(sid == 0)
        def _():
            pltpu.sync_copy(res2_v, out_h.at[cid])

    return body


def kernel(intr_idxs_ch, intr_divs_ch, feat_idxs_ch, smpl_segment_ids,
           vecs, intr_W, intr_b):
    T = intr_idxs_ch.shape[0]
    n_intrs, vec_size = intr_W.shape[0], vecs.shape[1]
    n_segs = 16
    fa = feat_idxs_ch[:, 0]
    fb = feat_idxs_ch[:, 1]
    wv = intr_W.reshape(-1)
    b16 = jnp.broadcast_to(intr_b, (L,))
    part = _sc_fm(T, n_intrs, vec_size, n_segs)(
        fa, fb, intr_idxs_ch, intr_divs_ch, smpl_segment_ids, vecs, wv, b16)
    return part[0] + part[1]


# speedup vs baseline: 1.1209x; 1.1209x over previous
import functools

import jax
import jax.numpy as jnp
from jax import lax
from jax.experimental import pallas as pl
from jax.experimental.pallas import tpu as pltpu
from jax.experimental.pallas import tpu_sc as plsc

L = 16
NC = 2
NS = 16
NW = NC * NS
CHUNK = 128
KUNROLL = 8


def _sc_fm(T, n_intrs, vec_size, n_segs):
    tpw = T // NW
    nchunk = tpw // CHUNK
    nhblk = (CHUNK // 2) // L

    mesh = plsc.VectorSubcoreMesh(
        core_axis_name="c", subcore_axis_name="s",
        num_cores=NC, num_subcores=NS)
    f32, i32 = jnp.float32, jnp.int32

    @functools.partial(
        pl.kernel,
        out_type=jax.ShapeDtypeStruct((NC, n_segs), f32),
        mesh=mesh,
        compiler_params=pltpu.CompilerParams(needs_layout_passes=False),
        scratch_types=dict(
            ia0=pltpu.VMEM((CHUNK,), i32),
            ia1=pltpu.VMEM((CHUNK,), i32),
            ia2=pltpu.VMEM((CHUNK,), i32),
            ib0=pltpu.VMEM((CHUNK,), i32),
            ib1=pltpu.VMEM((CHUNK,), i32),
            ib2=pltpu.VMEM((CHUNK,), i32),
            iv_v=pltpu.VMEM((tpw,), i32),
            dv_v=pltpu.VMEM((tpw,), f32),
            sg_v=pltpu.VMEM((tpw,), i32),
            wv_v=pltpu.VMEM((n_intrs,), f32),
            b_v=pltpu.VMEM((L,), f32),
            bufa0=pltpu.VMEM((CHUNK, vec_size), f32),
            bufa1=pltpu.VMEM((CHUNK, vec_size), f32),
            bufa2=pltpu.VMEM((CHUNK, vec_size), f32),
            bufb0=pltpu.VMEM((CHUNK, vec_size), f32),
            bufb1=pltpu.VMEM((CHUNK, vec_size), f32),
            bufb2=pltpu.VMEM((CHUNK, vec_size), f32),
            d_v=pltpu.VMEM((tpw,), f32),
            accm_v=pltpu.VMEM((n_segs * L,), f32),
            res_v=pltpu.VMEM((L,), f32),
            merge_v=pltpu.VMEM((NS, L), f32),
            res2_v=pltpu.VMEM((L,), f32),
            shared=pltpu.VMEM_SHARED((NS, L), f32),
            sem0=pltpu.SemaphoreType.DMA,
            sem1=pltpu.SemaphoreType.DMA,
            sem2=pltpu.SemaphoreType.DMA,
            sem3=pltpu.SemaphoreType.DMA,
        ),
    )
    def body(fa_h, fb_h, iv_h, dv_h, sg_h, vecs_h, wv_h, b_h, out_h,
             ia0, ia1, ia2, ib0, ib1, ib2, iv_v, dv_v, sg_v, wv_v, b_v,
             bufa0, bufa1, bufa2, bufb0, bufb1, bufb2, d_v, accm_v, res_v,
             merge_v, res2_v, shared, sem0, sem1, sem2, sem3):
        cid = lax.axis_index("c")
        sid = lax.axis_index("s")
        wid = cid * NS + sid
        base = wid * tpw

        bufs_a = (bufa0, bufa1, bufa2)
        bufs_b = (bufb0, bufb1, bufb2)
        ias = (ia0, ia1, ia2)
        ibs = (ib0, ib1, ib2)
        sems = (sem0, sem1, sem2)
        iota = lax.iota(i32, L)

        def issue(c):
            slot = c % 3
            off = base + c * CHUNK
            pltpu.sync_copy(fa_h.at[pl.ds(off, CHUNK)], ias[slot])
            pltpu.sync_copy(fb_h.at[pl.ds(off, CHUNK)], ibs[slot])
            da = pltpu.async_copy(vecs_h.at[ias[slot]], bufs_a[slot], sems[slot])
            db = pltpu.async_copy(vecs_h.at[ibs[slot]], bufs_b[slot], sems[slot])
            return (da, db)

        def compute(c, bufa, bufb):
            def blk_body(b, _):
                rows = b * L + iota

                def k_body(i, accs):
                    a0, a1 = accs
                    for u in range(KUNROLL):
                        kk = i * KUNROLL + u
                        col = (iota + kk) & (vec_size - 1)
                        va = plsc.load_gather(bufa, [rows, col])
                        vb = plsc.load_gather(bufb, [rows, col])
                        p = va * vb
                        if u % 2 == 0:
                            a0 = a0 + p
                        else:
                            a1 = a1 + p
                    return a0, a1

                z = jnp.zeros((L,), f32)
                a0, a1 = lax.fori_loop(0, vec_size // KUNROLL, k_body, (z, z))
                d_v[pl.ds(c * CHUNK + b * L, L)] = a0 + a1
                return 0

            lax.fori_loop(0, 2 * nhblk, blk_body, 0)

        descs = {0: issue(0), 1: issue(1)}
        st = [
            pltpu.async_copy(iv_h.at[pl.ds(base, tpw)], iv_v, sem3),
            pltpu.async_copy(dv_h.at[pl.ds(base, tpw)], dv_v, sem3),
            pltpu.async_copy(sg_h.at[pl.ds(base, tpw)], sg_v, sem3),
            pltpu.async_copy(wv_h, wv_v, sem3),
            pltpu.async_copy(b_h, b_v, sem3),
        ]
        accs = tuple(jnp.zeros((L,), f32) for _ in range(n_segs))
        bvec = None
        for c in range(nchunk):
            if c + 2 < nchunk:
                descs[c + 2] = issue(c + 2)
            for d_ in descs.pop(c):
                d_.wait()
            compute(c, bufs_a[c % 3], bufs_b[c % 3])
            if c == 0:
                for d_ in st:
                    d_.wait()
                bvec = b_v[...]

            def seg_body(j, accs, _c=c, _b=bvec):
                t0 = _c * CHUNK + j * L
                ivec = iv_v[pl.ds(t0, L)]
                w = plsc.load_gather(wv_v, [ivec])
                dv = dv_v[pl.ds(t0, L)]
                d = d_v[pl.ds(t0, L)]
                sg = sg_v[pl.ds(t0, L)]
                wdb = w / dv * d + _b
                return tuple(
                    accs[s] + jnp.where(sg == s, wdb, 0.0)
                    for s in range(n_segs))

            accs = lax.fori_loop(0, CHUNK // L, seg_body, accs)

        for s in range(n_segs):
            accm_v[pl.ds(s * L, L)] = accs[s]

        plsc.subcore_barrier()
        res = jnp.zeros((L,), f32)
        for l in range(L):
            res = res + plsc.load_gather(accm_v, [iota * L + l])
        res_v[...] = res
        plsc.subcore_barrier()

        pltpu.sync_copy(res_v, shared.at[sid])
        plsc.subcore_barrier()

        @pl.when(sid == 0)
        def _():
            pltpu.sync_copy(shared, merge_v)
            m = merge_v[0, :]
            for s in range(1, NS):
                m = m + merge_v[s, :]
            res2_v[...] = m

        plsc.subcore_barrier()

        @pl.when(sid == 0)
        def _():
            pltpu.sync_copy(res2_v, out_h.at[cid])

    return body


def kernel(intr_idxs_ch, intr_divs_ch, feat_idxs_ch, smpl_segment_ids,
           vecs, intr_W, intr_b):
    T = intr_idxs_ch.shape[0]
    n_intrs, vec_size = intr_W.shape[0], vecs.shape[1]
    n_segs = 16
    fa = feat_idxs_ch[:, 0]
    fb = feat_idxs_ch[:, 1]
    wv = intr_W.reshape(-1)
    b16 = jnp.broadcast_to(intr_b, (L,))
    part = _sc_fm(T, n_intrs, vec_size, n_segs)(
        fa, fb, intr_idxs_ch, intr_divs_ch, smpl_segment_ids, vecs, wv, b16)
    return part[0] + part[1]


# speedup vs baseline: 1.1300x; 1.0081x over previous
import functools

import jax
import jax.numpy as jnp
from jax import lax
from jax.experimental import pallas as pl
from jax.experimental.pallas import tpu as pltpu
from jax.experimental.pallas import tpu_sc as plsc

L = 16
NC = 2
NS = 16
NW = NC * NS
CHUNK = 128
KUNROLL = 8


def _sc_fm(T, n_intrs, vec_size, n_segs):
    tpw = T // NW
    nchunk = tpw // CHUNK
    nhblk = (CHUNK // 2) // L

    mesh = plsc.VectorSubcoreMesh(
        core_axis_name="c", subcore_axis_name="s",
        num_cores=NC, num_subcores=NS)
    f32, i32 = jnp.float32, jnp.int32

    @functools.partial(
        pl.kernel,
        out_type=jax.ShapeDtypeStruct((NC, n_segs), f32),
        mesh=mesh,
        compiler_params=pltpu.CompilerParams(needs_layout_passes=False),
        scratch_types=dict(
            ia0=pltpu.VMEM((CHUNK,), i32),
            ia1=pltpu.VMEM((CHUNK,), i32),
            ia2=pltpu.VMEM((CHUNK,), i32),
            ib0=pltpu.VMEM((CHUNK,), i32),
            ib1=pltpu.VMEM((CHUNK,), i32),
            ib2=pltpu.VMEM((CHUNK,), i32),
            iv_v=pltpu.VMEM((tpw,), i32),
            dv_v=pltpu.VMEM((tpw,), f32),
            sg_v=pltpu.VMEM((tpw,), i32),
            wv_v=pltpu.VMEM((n_intrs,), f32),
            b_v=pltpu.VMEM((L,), f32),
            bufa0=pltpu.VMEM((CHUNK, vec_size), f32),
            bufa1=pltpu.VMEM((CHUNK, vec_size), f32),
            bufa2=pltpu.VMEM((CHUNK, vec_size), f32),
            bufb0=pltpu.VMEM((CHUNK, vec_size), f32),
            bufb1=pltpu.VMEM((CHUNK, vec_size), f32),
            bufb2=pltpu.VMEM((CHUNK, vec_size), f32),
            d_v=pltpu.VMEM((tpw,), f32),
            accm_v=pltpu.VMEM((n_segs * L,), f32),
            res_v=pltpu.VMEM((L,), f32),
            merge_v=pltpu.VMEM((NS, L), f32),
            res2_v=pltpu.VMEM((L,), f32),
            shared=pltpu.VMEM_SHARED((NS, L), f32),
            sem0=pltpu.SemaphoreType.DMA,
            sem1=pltpu.SemaphoreType.DMA,
            sem2=pltpu.SemaphoreType.DMA,
            sem3=pltpu.SemaphoreType.DMA,
        ),
    )
    def body(fa_h, fb_h, iv_h, dv_h, sg_h, vecs_h, wv_h, b_h, out_h,
             ia0, ia1, ia2, ib0, ib1, ib2, iv_v, dv_v, sg_v, wv_v, b_v,
             bufa0, bufa1, bufa2, bufb0, bufb1, bufb2, d_v, accm_v, res_v,
             merge_v, res2_v, shared, sem0, sem1, sem2, sem3):
        cid = lax.axis_index("c")
        sid = lax.axis_index("s")
        wid = cid * NS + sid
        base = wid * tpw

        bufs_a = (bufa0, bufa1, bufa2)
        bufs_b = (bufb0, bufb1, bufb2)
        ias = (ia0, ia1, ia2)
        ibs = (ib0, ib1, ib2)
        sems = (sem0, sem1, sem2)
        iota = lax.iota(i32, L)

        def issue(c):
            slot = c % 3
            off = base + c * CHUNK
            pltpu.sync_copy(fa_h.at[pl.ds(off, CHUNK)], ias[slot])
            pltpu.sync_copy(fb_h.at[pl.ds(off, CHUNK)], ibs[slot])
            da = pltpu.async_copy(vecs_h.at[ias[slot]], bufs_a[slot], sems[slot])
            db = pltpu.async_copy(vecs_h.at[ibs[slot]], bufs_b[slot], sems[slot])
            return (da, db)

        def compute(c, bufa, bufb):
            def blk_body(b, _):
                rows = b * L + iota

                def k_body(i, accs):
                    a0, a1 = accs
                    for u in range(KUNROLL):
                        kk = i * KUNROLL + u
                        col = (iota + kk) & (vec_size - 1)
                        va = plsc.load_gather(bufa, [rows, col])
                        vb = plsc.load_gather(bufb, [rows, col])
                        p = va * vb
                        if u % 2 == 0:
                            a0 = a0 + p
                        else:
                            a1 = a1 + p
                    return a0, a1

                z = jnp.zeros((L,), f32)
                a0, a1 = lax.fori_loop(0, vec_size // KUNROLL, k_body, (z, z))
                d_v[pl.ds(c * CHUNK + b * L, L)] = a0 + a1
                return 0

            lax.fori_loop(0, 2 * nhblk, blk_body, 0)

        descs = {0: issue(0), 1: issue(1)}
        st = [
            pltpu.async_copy(iv_h.at[pl.ds(base, tpw)], iv_v, sem3),
            pltpu.async_copy(dv_h.at[pl.ds(base, tpw)], dv_v, sem3),
            pltpu.async_copy(sg_h.at[pl.ds(base, tpw)], sg_v, sem3),
            pltpu.async_copy(wv_h, wv_v, sem3),
            pltpu.async_copy(b_h, b_v, sem3),
        ]
        for c in range(nchunk):
            if c + 2 < nchunk:
                descs[c + 2] = issue(c + 2)
            for d_ in descs.pop(c):
                d_.wait()
            compute(c, bufs_a[c % 3], bufs_b[c % 3])

        for d_ in st:
            d_.wait()
        plsc.subcore_barrier()
        bvec = b_v[...]

        def seg_body(i, accs):
            t0 = i * L
            ivec = iv_v[pl.ds(t0, L)]
            w = plsc.load_gather(wv_v, [ivec])
            dv = dv_v[pl.ds(t0, L)]
            d = d_v[pl.ds(t0, L)]
            sg = sg_v[pl.ds(t0, L)]
            wdb = w / dv * d + bvec
            return tuple(
                accs[s] + jnp.where(sg == s, wdb, 0.0) for s in range(n_segs))

        zeros = tuple(jnp.zeros((L,), f32) for _ in range(n_segs))
        accs = lax.fori_loop(0, tpw // L, seg_body, zeros)
        for s in range(n_segs):
            accm_v[pl.ds(s * L, L)] = accs[s]

        plsc.subcore_barrier()
        res = jnp.zeros((L,), f32)
        for l in range(L):
            res = res + plsc.load_gather(accm_v, [iota * L + l])
        res_v[...] = res
        plsc.subcore_barrier()

        pltpu.sync_copy(res_v, shared.at[sid])
        plsc.subcore_barrier()

        @pl.when(sid == 0)
        def _():
            pltpu.sync_copy(shared, merge_v)
            m = merge_v[0, :]
            for s in range(1, NS):
                m = m + merge_v[s, :]
            res2_v[...] = m

        plsc.subcore_barrier()

        @pl.when(sid == 0)
        def _():
            pltpu.sync_copy(res2_v, out_h.at[cid])

    return body


def kernel(intr_idxs_ch, intr_divs_ch, feat_idxs_ch, smpl_segment_ids,
           vecs, intr_W, intr_b):
    T = intr_idxs_ch.shape[0]
    n_intrs, vec_size = intr_W.shape[0], vecs.shape[1]
    n_segs = 16
    fa = feat_idxs_ch[:, 0]
    fb = feat_idxs_ch[:, 1]
    wv = intr_W.reshape(-1)
    b16 = jnp.broadcast_to(intr_b, (L,))
    part = _sc_fm(T, n_intrs, vec_size, n_segs)(
        fa, fb, intr_idxs_ch, intr_divs_ch, smpl_segment_ids, vecs, wv, b16)
    return part[0] + part[1]
